# Initial kernel scaffold; baseline (speedup 1.0000x reference)
import jax
import jax.numpy as jnp
from jax.experimental import pallas as pl


def kernel(x, e, edge_index0, edge_index1, pos_edges, neg_edges, W1_0, b1_0, W2_0, b2_0, W1_1, b1_1, W2_1, b2_1, Wp1, bp1, Wp2, bp2):
    raise NotImplementedError("write your pallas kernel here")


# speedup vs baseline: 2.3916x; 2.3916x over previous
import functools

import jax
import jax.numpy as jnp
from jax import lax
from jax.experimental import pallas as pl
from jax.experimental.pallas import tpu as pltpu
from jax.experimental.pallas import tpu_sc as plsc

N = 10000
E = 320000
D = 128
PE = 10000

NC = 2
NS = 16
NW = NC * NS

CHUNK = 128
E_PAD = 327680
EDGES_PER_W = E_PAD // NW
N_CHUNKS = EDGES_PER_W // CHUNK
DUMMY_ROW = N
NP = 10112
ROWS_PER_TILE = NP // NS

PE_PAD = 10240
G_TOTAL = 4 * PE_PAD
G_PER_W = G_TOTAL // NW
G_CHUNKS = G_PER_W // CHUNK

_sc_mesh = plsc.VectorSubcoreMesh(core_axis_name="c", subcore_axis_name="s")


@functools.partial(
    pl.kernel,
    out_type=jax.ShapeDtypeStruct((NC, NP, D), jnp.float32),
    mesh=_sc_mesh,
    scratch_types=[
        pltpu.VMEM_SHARED((NP, D), jnp.float32),
        pltpu.VMEM((CHUNK,), jnp.int32),
        pltpu.VMEM((CHUNK,), jnp.int32),
        pltpu.VMEM((CHUNK, D), jnp.float32),
        pltpu.SemaphoreType.DMA,
    ],
)
def _sc_scatter_add(h_hbm, src_hbm, dst_hbm, zeros_hbm, out_hbm,
                    acc_sh, sidx_v, didx_v, rows_v, sem):
    cid = lax.axis_index("c")
    sid = lax.axis_index("s")
    wid = sid * NC + cid

    pltpu.sync_copy(zeros_hbm.at[pl.ds(sid * ROWS_PER_TILE, ROWS_PER_TILE)],
                    acc_sh.at[pl.ds(sid * ROWS_PER_TILE, ROWS_PER_TILE)])
    plsc.subcore_barrier()

    base = wid * EDGES_PER_W

    def body(c, carry):
        off = base + c * CHUNK
        pltpu.sync_copy(src_hbm.at[pl.ds(off, CHUNK)], sidx_v)
        pltpu.sync_copy(dst_hbm.at[pl.ds(off, CHUNK)], didx_v)
        pltpu.async_copy(h_hbm.at[sidx_v], rows_v, sem).wait()
        pltpu.sync_copy(rows_v, acc_sh.at[didx_v], add=True)
        return carry

    lax.fori_loop(0, N_CHUNKS, body, 0)
    plsc.subcore_barrier()

    pltpu.sync_copy(acc_sh.at[pl.ds(sid * ROWS_PER_TILE, ROWS_PER_TILE)],
                    out_hbm.at[cid].at[pl.ds(sid * ROWS_PER_TILE, ROWS_PER_TILE)])


@functools.partial(
    pl.kernel,
    out_type=jax.ShapeDtypeStruct((G_TOTAL, D), jnp.float32),
    mesh=_sc_mesh,
    scratch_types=[
        pltpu.VMEM((CHUNK,), jnp.int32),
        pltpu.VMEM((CHUNK, D), jnp.float32),
        pltpu.SemaphoreType.DMA,
    ],
)
def _sc_gather(h_hbm, idx_hbm, out_hbm, idx_v, rows_v, sem):
    cid = lax.axis_index("c")
    sid = lax.axis_index("s")
    wid = sid * NC + cid
    base = wid * G_PER_W

    def body(c, carry):
        off = base + c * CHUNK
        pltpu.sync_copy(idx_hbm.at[pl.ds(off, CHUNK)], idx_v)
        pltpu.async_copy(h_hbm.at[idx_v], rows_v, sem).wait()
        pltpu.sync_copy(rows_v, out_hbm.at[pl.ds(off, CHUNK)])
        return carry

    lax.fori_loop(0, G_CHUNKS, body, 0)


def _tc_mlp_body(h_ref, a0_ref, a1_ref, w1_ref, b1_ref, w2_ref, b2_ref, o_ref):
    z = h_ref[...] + a0_ref[0] + a1_ref[0]
    y = jnp.maximum(
        jnp.dot(z, w1_ref[...], preferred_element_type=jnp.float32)
        + b1_ref[...], 0.0)
    o = jnp.dot(y, w2_ref[...], preferred_element_type=jnp.float32) + b2_ref[...]
    o_ref[...] = jnp.maximum(o, 0.0)


def _tc_mlp(h, agg, w1, b1, w2, b2):
    blk = 1000
    grid = (N // blk,)
    return pl.pallas_call(
        _tc_mlp_body,
        grid=grid,
        in_specs=[
            pl.BlockSpec((blk, D), lambda i: (i, 0)),
            pl.BlockSpec((1, blk, D), lambda i: (0, i, 0)),
            pl.BlockSpec((1, blk, D), lambda i: (1, i, 0)),
            pl.BlockSpec((D, D), lambda i: (0, 0)),
            pl.BlockSpec((1, D), lambda i: (0, 0)),
            pl.BlockSpec((D, D), lambda i: (0, 0)),
            pl.BlockSpec((1, D), lambda i: (0, 0)),
        ],
        out_specs=pl.BlockSpec((blk, D), lambda i: (i, 0)),
        out_shape=jax.ShapeDtypeStruct((N, D), jnp.float32),
    )(h, agg, agg, w1, b1.reshape(1, D), w2, b2.reshape(1, D))


def _tc_pred_body(a_ref, b_ref, w1_ref, b1_ref, w2_ref, b2_ref, o_ref):
    p = a_ref[...] * b_ref[...]
    y = jnp.maximum(
        jnp.dot(p, w1_ref[...], preferred_element_type=jnp.float32)
        + b1_ref[...], 0.0)
    o_ref[...] = (jnp.dot(y, w2_ref[...], preferred_element_type=jnp.float32)
                  + b2_ref[...])


def _tc_pred(g, wp1, bp1, wp2, bp2):
    blk = 1024
    nblk = PE_PAD // blk
    grid = (2 * nblk,)
    return pl.pallas_call(
        _tc_pred_body,
        grid=grid,
        in_specs=[
            pl.BlockSpec((blk, D), lambda i: ((i // nblk) * 2 * nblk + i % nblk, 0)),
            pl.BlockSpec((blk, D), lambda i: ((i // nblk) * 2 * nblk + nblk + i % nblk, 0)),
            pl.BlockSpec((D, D // 2), lambda i: (0, 0)),
            pl.BlockSpec((1, D // 2), lambda i: (0, 0)),
            pl.BlockSpec((D // 2, 1), lambda i: (0, 0)),
            pl.BlockSpec((1, 1), lambda i: (0, 0)),
        ],
        out_specs=pl.BlockSpec((blk, 1), lambda i: (i, 0)),
        out_shape=jax.ShapeDtypeStruct((2 * PE_PAD, 1), jnp.float32),
    )(g, g, wp1, bp1.reshape(1, D // 2), wp2, bp2.reshape(1, 1))


def kernel(x, e, edge_index0, edge_index1, pos_edges, neg_edges,
           W1_0, b1_0, W2_0, b2_0, W1_1, b1_1, W2_1, b2_1,
           Wp1, bp1, Wp2, bp2):
    del e
    zeros = jnp.zeros((NP, D), jnp.float32)
    pad_src = jnp.zeros((E_PAD - E,), jnp.int32)
    pad_dst = jnp.full((E_PAD - E,), DUMMY_ROW, jnp.int32)

    src0 = jnp.concatenate([edge_index0[0], pad_src])
    dst0 = jnp.concatenate([edge_index0[1], pad_dst])
    src1 = jnp.concatenate([edge_index1[0], pad_src])
    dst1 = jnp.concatenate([edge_index1[1], pad_dst])

    agg0 = _sc_scatter_add(x, src0, dst0, zeros)
    h1 = _tc_mlp(x, agg0, W1_0, b1_0, W2_0, b2_0)

    agg1 = _sc_scatter_add(h1, src1, dst1, zeros)
    h2 = _tc_mlp(h1, agg1, W1_1, b1_1, W2_1, b2_1)

    gpad = jnp.zeros((PE_PAD - PE,), jnp.int32)
    idx_all = jnp.concatenate([
        pos_edges[0], gpad, pos_edges[1], gpad,
        neg_edges[0], gpad, neg_edges[1], gpad,
    ])
    g = _sc_gather(h2, idx_all)
    pred = _tc_pred(g, Wp1, bp1, Wp2, bp2)
    h_pos = pred[:PE]
    h_neg = pred[PE_PAD:PE_PAD + PE]
    return (h_pos, h_neg, h2)


# speedup vs baseline: 2.9239x; 1.2226x over previous
import functools

import jax
import jax.numpy as jnp
from jax import lax
from jax.experimental import pallas as pl
from jax.experimental.pallas import tpu as pltpu
from jax.experimental.pallas import tpu_sc as plsc

N = 10000
E = 320000
D = 128
PE = 10000

NC = 2
NS = 16
NW = NC * NS

CHUNK = 128
E_PAD = 327680
EDGES_PER_W = E_PAD // NW
N_CHUNKS = EDGES_PER_W // CHUNK
DUMMY_ROW = N
NP = 10112
ROWS_PER_TILE = NP // NS

PE_PAD = 10240
G_TOTAL = 4 * PE_PAD
G_PER_W = G_TOTAL // NW
G_CHUNKS = G_PER_W // CHUNK

_sc_mesh = plsc.VectorSubcoreMesh(core_axis_name="c", subcore_axis_name="s")


NBUF = 2
NBUF_G = 5
NSLAB = 40


@functools.partial(
    pl.kernel,
    out_type=jax.ShapeDtypeStruct((NC, NP, D), jnp.float32),
    mesh=_sc_mesh,
    scratch_types=[
        pltpu.VMEM_SHARED((NP, D), jnp.float32),
        pltpu.VMEM((NSLAB, CHUNK), jnp.int32),
        pltpu.VMEM((NSLAB, CHUNK), jnp.int32),
        [pltpu.VMEM((CHUNK, D), jnp.float32) for _ in range(NBUF)],
        [pltpu.SemaphoreType.DMA for _ in range(NBUF)],
    ],
)
def _sc_scatter_add(h_hbm, src_hbm, dst_hbm, zeros_hbm, out_hbm,
                    acc_sh, sidx_v, didx_v, rows, sems):
    cid = lax.axis_index("c")
    sid = lax.axis_index("s")
    wid = sid * NC + cid

    pltpu.sync_copy(zeros_hbm.at[pl.ds(sid * ROWS_PER_TILE, ROWS_PER_TILE)],
                    acc_sh.at[pl.ds(sid * ROWS_PER_TILE, ROWS_PER_TILE)])
    plsc.subcore_barrier()

    def start(c, b):
        pltpu.async_copy(h_hbm.at[sidx_v.at[c]], rows[b], sems[b])

    def finish(c, b):
        pltpu.make_async_copy(h_hbm.at[sidx_v.at[c]], rows[b], sems[b]).wait()
        pltpu.sync_copy(rows[b], acc_sh.at[didx_v.at[c]], add=True)

    for p in range(N_CHUNKS // NSLAB):
        row0 = wid * N_CHUNKS + p * NSLAB
        pltpu.sync_copy(src_hbm.at[pl.ds(row0, NSLAB)], sidx_v)
        pltpu.sync_copy(dst_hbm.at[pl.ds(row0, NSLAB)], didx_v)

        for b in range(NBUF):
            start(b, b)

        def body(j, carry):
            c = j * NBUF
            for b in range(NBUF):
                finish(c + b, b)
                start(c + NBUF + b, b)
            return carry

        lax.fori_loop(0, NSLAB // NBUF - 1, body, 0)
        c0 = NSLAB - NBUF
        for b in range(NBUF):
            finish(c0 + b, b)

    plsc.subcore_barrier()

    pltpu.sync_copy(acc_sh.at[pl.ds(sid * ROWS_PER_TILE, ROWS_PER_TILE)],
                    out_hbm.at[cid].at[pl.ds(sid * ROWS_PER_TILE, ROWS_PER_TILE)])


@functools.partial(
    pl.kernel,
    out_type=jax.ShapeDtypeStruct((G_TOTAL, D), jnp.float32),
    mesh=_sc_mesh,
    scratch_types=[
        pltpu.VMEM((G_PER_W,), jnp.int32),
        [pltpu.VMEM((CHUNK, D), jnp.float32) for _ in range(NBUF_G)],
        [pltpu.SemaphoreType.DMA for _ in range(NBUF_G)],
    ],
)
def _sc_gather(h_hbm, idx_hbm, out_hbm, idx_v, rows, sems):
    cid = lax.axis_index("c")
    sid = lax.axis_index("s")
    wid = sid * NC + cid
    base = wid * G_PER_W

    pltpu.sync_copy(idx_hbm.at[pl.ds(base, G_PER_W)], idx_v)

    def start(c, b):
        pltpu.async_copy(h_hbm.at[idx_v.at[pl.ds(c * CHUNK, CHUNK)]],
                         rows[b], sems[b])

    def finish(c, b):
        pltpu.make_async_copy(h_hbm.at[idx_v.at[pl.ds(c * CHUNK, CHUNK)]],
                              rows[b], sems[b]).wait()
        pltpu.sync_copy(rows[b], out_hbm.at[pl.ds(base + c * CHUNK, CHUNK)])

    for b in range(NBUF_G):
        start(b, b)

    def body(j, carry):
        c = j * NBUF_G
        for b in range(NBUF_G):
            finish(c + b, b)
            start(c + NBUF_G + b, b)
        return carry

    lax.fori_loop(0, G_CHUNKS // NBUF_G - 1, body, 0)
    c0 = G_CHUNKS - NBUF_G
    for b in range(NBUF_G):
        finish(c0 + b, b)


def _tc_mlp_body(h_ref, a0_ref, a1_ref, w1_ref, b1_ref, w2_ref, b2_ref, o_ref):
    z = h_ref[...] + a0_ref[0] + a1_ref[0]
    y = jnp.maximum(
        jnp.dot(z, w1_ref[...], preferred_element_type=jnp.float32)
        + b1_ref[...], 0.0)
    o = jnp.dot(y, w2_ref[...], preferred_element_type=jnp.float32) + b2_ref[...]
    o_ref[...] = jnp.maximum(o, 0.0)


def _tc_mlp(h, agg, w1, b1, w2, b2):
    blk = 1000
    grid = (N // blk,)
    return pl.pallas_call(
        _tc_mlp_body,
        grid=grid,
        in_specs=[
            pl.BlockSpec((blk, D), lambda i: (i, 0)),
            pl.BlockSpec((1, blk, D), lambda i: (0, i, 0)),
            pl.BlockSpec((1, blk, D), lambda i: (1, i, 0)),
            pl.BlockSpec((D, D), lambda i: (0, 0)),
            pl.BlockSpec((1, D), lambda i: (0, 0)),
            pl.BlockSpec((D, D), lambda i: (0, 0)),
            pl.BlockSpec((1, D), lambda i: (0, 0)),
        ],
        out_specs=pl.BlockSpec((blk, D), lambda i: (i, 0)),
        out_shape=jax.ShapeDtypeStruct((N, D), jnp.float32),
    )(h, agg, agg, w1, b1.reshape(1, D), w2, b2.reshape(1, D))


def _tc_pred_body(a_ref, b_ref, w1_ref, b1_ref, w2_ref, b2_ref, o_ref):
    p = a_ref[...] * b_ref[...]
    y = jnp.maximum(
        jnp.dot(p, w1_ref[...], preferred_element_type=jnp.float32)
        + b1_ref[...], 0.0)
    o_ref[...] = (jnp.dot(y, w2_ref[...], preferred_element_type=jnp.float32)
                  + b2_ref[...])


def _tc_pred(g, wp1, bp1, wp2, bp2):
    blk = 1024
    nblk = PE_PAD // blk
    grid = (2 * nblk,)
    return pl.pallas_call(
        _tc_pred_body,
        grid=grid,
        in_specs=[
            pl.BlockSpec((blk, D), lambda i: ((i // nblk) * 2 * nblk + i % nblk, 0)),
            pl.BlockSpec((blk, D), lambda i: ((i // nblk) * 2 * nblk + nblk + i % nblk, 0)),
            pl.BlockSpec((D, D // 2), lambda i: (0, 0)),
            pl.BlockSpec((1, D // 2), lambda i: (0, 0)),
            pl.BlockSpec((D // 2, 1), lambda i: (0, 0)),
            pl.BlockSpec((1, 1), lambda i: (0, 0)),
        ],
        out_specs=pl.BlockSpec((blk, 1), lambda i: (i, 0)),
        out_shape=jax.ShapeDtypeStruct((2 * PE_PAD, 1), jnp.float32),
    )(g, g, wp1, bp1.reshape(1, D // 2), wp2, bp2.reshape(1, 1))


def kernel(x, e, edge_index0, edge_index1, pos_edges, neg_edges,
           W1_0, b1_0, W2_0, b2_0, W1_1, b1_1, W2_1, b2_1,
           Wp1, bp1, Wp2, bp2):
    del e
    zeros = jnp.zeros((NP, D), jnp.float32)
    pad_src = jnp.zeros((E_PAD - E,), jnp.int32)
    pad_dst = jnp.full((E_PAD - E,), DUMMY_ROW, jnp.int32)

    src0 = jnp.concatenate([edge_index0[0], pad_src]).reshape(-1, CHUNK)
    dst0 = jnp.concatenate([edge_index0[1], pad_dst]).reshape(-1, CHUNK)
    src1 = jnp.concatenate([edge_index1[0], pad_src]).reshape(-1, CHUNK)
    dst1 = jnp.concatenate([edge_index1[1], pad_dst]).reshape(-1, CHUNK)

    agg0 = _sc_scatter_add(x, src0, dst0, zeros)
    h1 = _tc_mlp(x, agg0, W1_0, b1_0, W2_0, b2_0)

    agg1 = _sc_scatter_add(h1, src1, dst1, zeros)
    h2 = _tc_mlp(h1, agg1, W1_1, b1_1, W2_1, b2_1)

    gpad = jnp.zeros((PE_PAD - PE,), jnp.int32)
    idx_all = jnp.concatenate([
        pos_edges[0], gpad, pos_edges[1], gpad,
        neg_edges[0], gpad, neg_edges[1], gpad,
    ])
    g = _sc_gather(h2, idx_all)
    pred = _tc_pred(g, Wp1, bp1, Wp2, bp2)
    h_pos = pred[:PE]
    h_neg = pred[PE_PAD:PE_PAD + PE]
    return (h_pos, h_neg, h2)


# speedup vs baseline: 2.9264x; 1.0009x over previous
import functools

import jax
import jax.numpy as jnp
from jax import lax
from jax.experimental import pallas as pl
from jax.experimental.pallas import tpu as pltpu
from jax.experimental.pallas import tpu_sc as plsc

N = 10000
E = 320000
D = 128
PE = 10000

NC = 2
NS = 16
NW = NC * NS

CHUNK = 128
E_PAD = 327680
EDGES_PER_W = E_PAD // NW
N_CHUNKS = EDGES_PER_W // CHUNK
DUMMY_ROW = N
NP = 10112
ROWS_PER_TILE = NP // NS

PE_PAD = 10240
G_TOTAL = 4 * PE_PAD
G_PER_W = G_TOTAL // NW
G_CHUNKS = G_PER_W // CHUNK

_sc_mesh = plsc.VectorSubcoreMesh(core_axis_name="c", subcore_axis_name="s")


NBUF = 2
NBUF_G = 5
NSLAB = 40


@functools.partial(
    pl.kernel,
    out_type=jax.ShapeDtypeStruct((NC, NP, D), jnp.float32),
    mesh=_sc_mesh,
    scratch_types=[
        pltpu.VMEM_SHARED((NP, D), jnp.float32),
        pltpu.VMEM((NSLAB, CHUNK), jnp.int32),
        pltpu.VMEM((NSLAB, CHUNK), jnp.int32),
        [pltpu.VMEM((CHUNK, D), jnp.float32) for _ in range(NBUF)],
        [pltpu.SemaphoreType.DMA for _ in range(NBUF)],
    ],
)
def _sc_scatter_add(h_hbm, src_hbm, dst_hbm, zeros_hbm, out_hbm,
                    acc_sh, sidx_v, didx_v, rows, sems):
    cid = lax.axis_index("c")
    sid = lax.axis_index("s")
    wid = sid * NC + cid

    pltpu.sync_copy(zeros_hbm.at[pl.ds(sid * ROWS_PER_TILE, ROWS_PER_TILE)],
                    acc_sh.at[pl.ds(sid * ROWS_PER_TILE, ROWS_PER_TILE)])
    plsc.subcore_barrier()

    SPLIT = 4
    SUB = CHUNK // SPLIT

    def start(c, b):
        for k in range(SPLIT):
            pltpu.async_copy(h_hbm.at[sidx_v.at[c, pl.ds(k * SUB, SUB)]],
                             rows[b].at[pl.ds(k * SUB, SUB)], sems[b])

    def finish(c, b):
        pltpu.make_async_copy(h_hbm.at[sidx_v.at[c]], rows[b], sems[b]).wait()
        pltpu.sync_copy(rows[b], acc_sh.at[didx_v.at[c]], add=True)

    for p in range(N_CHUNKS // NSLAB):
        row0 = wid * N_CHUNKS + p * NSLAB
        pltpu.sync_copy(src_hbm.at[pl.ds(row0, NSLAB)], sidx_v)
        pltpu.sync_copy(dst_hbm.at[pl.ds(row0, NSLAB)], didx_v)

        for b in range(NBUF):
            start(b, b)

        def body(j, carry):
            c = j * NBUF
            for b in range(NBUF):
                finish(c + b, b)
                start(c + NBUF + b, b)
            return carry

        lax.fori_loop(0, NSLAB // NBUF - 1, body, 0)
        c0 = NSLAB - NBUF
        for b in range(NBUF):
            finish(c0 + b, b)

    plsc.subcore_barrier()

    pltpu.sync_copy(acc_sh.at[pl.ds(sid * ROWS_PER_TILE, ROWS_PER_TILE)],
                    out_hbm.at[cid].at[pl.ds(sid * ROWS_PER_TILE, ROWS_PER_TILE)])


@functools.partial(
    pl.kernel,
    out_type=jax.ShapeDtypeStruct((G_TOTAL, D), jnp.float32),
    mesh=_sc_mesh,
    scratch_types=[
        pltpu.VMEM((G_PER_W,), jnp.int32),
        [pltpu.VMEM((CHUNK, D), jnp.float32) for _ in range(NBUF_G)],
        [pltpu.SemaphoreType.DMA for _ in range(NBUF_G)],
    ],
)
def _sc_gather(h_hbm, idx_hbm, out_hbm, idx_v, rows, sems):
    cid = lax.axis_index("c")
    sid = lax.axis_index("s")
    wid = sid * NC + cid
    base = wid * G_PER_W

    pltpu.sync_copy(idx_hbm.at[pl.ds(base, G_PER_W)], idx_v)

    def start(c, b):
        pltpu.async_copy(h_hbm.at[idx_v.at[pl.ds(c * CHUNK, CHUNK)]],
                         rows[b], sems[b])

    def finish(c, b):
        pltpu.make_async_copy(h_hbm.at[idx_v.at[pl.ds(c * CHUNK, CHUNK)]],
                              rows[b], sems[b]).wait()
        pltpu.sync_copy(rows[b], out_hbm.at[pl.ds(base + c * CHUNK, CHUNK)])

    for b in range(NBUF_G):
        start(b, b)

    def body(j, carry):
        c = j * NBUF_G
        for b in range(NBUF_G):
            finish(c + b, b)
            start(c + NBUF_G + b, b)
        return carry

    lax.fori_loop(0, G_CHUNKS // NBUF_G - 1, body, 0)
    c0 = G_CHUNKS - NBUF_G
    for b in range(NBUF_G):
        finish(c0 + b, b)


def _tc_mlp_body(h_ref, a0_ref, a1_ref, w1_ref, b1_ref, w2_ref, b2_ref, o_ref):
    z = h_ref[...] + a0_ref[0] + a1_ref[0]
    y = jnp.maximum(
        jnp.dot(z, w1_ref[...], preferred_element_type=jnp.float32)
        + b1_ref[...], 0.0)
    o = jnp.dot(y, w2_ref[...], preferred_element_type=jnp.float32) + b2_ref[...]
    o_ref[...] = jnp.maximum(o, 0.0)


def _tc_mlp(h, agg, w1, b1, w2, b2):
    blk = 1000
    grid = (N // blk,)
    return pl.pallas_call(
        _tc_mlp_body,
        grid=grid,
        in_specs=[
            pl.BlockSpec((blk, D), lambda i: (i, 0)),
            pl.BlockSpec((1, blk, D), lambda i: (0, i, 0)),
            pl.BlockSpec((1, blk, D), lambda i: (1, i, 0)),
            pl.BlockSpec((D, D), lambda i: (0, 0)),
            pl.BlockSpec((1, D), lambda i: (0, 0)),
            pl.BlockSpec((D, D), lambda i: (0, 0)),
            pl.BlockSpec((1, D), lambda i: (0, 0)),
        ],
        out_specs=pl.BlockSpec((blk, D), lambda i: (i, 0)),
        out_shape=jax.ShapeDtypeStruct((N, D), jnp.float32),
    )(h, agg, agg, w1, b1.reshape(1, D), w2, b2.reshape(1, D))


def _tc_pred_body(a_ref, b_ref, w1_ref, b1_ref, w2_ref, b2_ref, o_ref):
    p = a_ref[...] * b_ref[...]
    y = jnp.maximum(
        jnp.dot(p, w1_ref[...], preferred_element_type=jnp.float32)
        + b1_ref[...], 0.0)
    o_ref[...] = (jnp.dot(y, w2_ref[...], preferred_element_type=jnp.float32)
                  + b2_ref[...])


def _tc_pred(g, wp1, bp1, wp2, bp2):
    blk = 1024
    nblk = PE_PAD // blk
    grid = (2 * nblk,)
    return pl.pallas_call(
        _tc_pred_body,
        grid=grid,
        in_specs=[
            pl.BlockSpec((blk, D), lambda i: ((i // nblk) * 2 * nblk + i % nblk, 0)),
            pl.BlockSpec((blk, D), lambda i: ((i // nblk) * 2 * nblk + nblk + i % nblk, 0)),
            pl.BlockSpec((D, D // 2), lambda i: (0, 0)),
            pl.BlockSpec((1, D // 2), lambda i: (0, 0)),
            pl.BlockSpec((D // 2, 1), lambda i: (0, 0)),
            pl.BlockSpec((1, 1), lambda i: (0, 0)),
        ],
        out_specs=pl.BlockSpec((blk, 1), lambda i: (i, 0)),
        out_shape=jax.ShapeDtypeStruct((2 * PE_PAD, 1), jnp.float32),
    )(g, g, wp1, bp1.reshape(1, D // 2), wp2, bp2.reshape(1, 1))


def kernel(x, e, edge_index0, edge_index1, pos_edges, neg_edges,
           W1_0, b1_0, W2_0, b2_0, W1_1, b1_1, W2_1, b2_1,
           Wp1, bp1, Wp2, bp2):
    del e
    zeros = jnp.zeros((NP, D), jnp.float32)
    pad_src = jnp.zeros((E_PAD - E,), jnp.int32)
    pad_dst = jnp.full((E_PAD - E,), DUMMY_ROW, jnp.int32)

    src0 = jnp.concatenate([edge_index0[0], pad_src]).reshape(-1, CHUNK)
    dst0 = jnp.concatenate([edge_index0[1], pad_dst]).reshape(-1, CHUNK)
    src1 = jnp.concatenate([edge_index1[0], pad_src]).reshape(-1, CHUNK)
    dst1 = jnp.concatenate([edge_index1[1], pad_dst]).reshape(-1, CHUNK)

    agg0 = _sc_scatter_add(x, src0, dst0, zeros)
    h1 = _tc_mlp(x, agg0, W1_0, b1_0, W2_0, b2_0)

    agg1 = _sc_scatter_add(h1, src1, dst1, zeros)
    h2 = _tc_mlp(h1, agg1, W1_1, b1_1, W2_1, b2_1)

    gpad = jnp.zeros((PE_PAD - PE,), jnp.int32)
    idx_all = jnp.concatenate([
        pos_edges[0], gpad, pos_edges[1], gpad,
        neg_edges[0], gpad, neg_edges[1], gpad,
    ])
    g = _sc_gather(h2, idx_all)
    pred = _tc_pred(g, Wp1, bp1, Wp2, bp2)
    h_pos = pred[:PE]
    h_neg = pred[PE_PAD:PE_PAD + PE]
    return (h_pos, h_neg, h2)


# speedup vs baseline: 3.0237x; 1.0332x over previous
import functools

import jax
import jax.numpy as jnp
from jax import lax
from jax.experimental import pallas as pl
from jax.experimental.pallas import tpu as pltpu
from jax.experimental.pallas import tpu_sc as plsc

N = 10000
E = 320000
D = 128
PE = 10000

NC = 2
NS = 16
NW = NC * NS

CHUNK = 128
E_PAD = 327680
EDGES_PER_W = E_PAD // NW
N_CHUNKS = EDGES_PER_W // CHUNK
DUMMY_ROW = N
NP = 10112
ROWS_PER_TILE = NP // NS

PE_PAD = 10240
G_TOTAL = 4 * PE_PAD
G_PER_W = G_TOTAL // NW
G_CHUNKS = G_PER_W // CHUNK

_sc_mesh = plsc.VectorSubcoreMesh(core_axis_name="c", subcore_axis_name="s")


NBUF = 2
NBUF_G = 5
NSLAB = 40


@functools.partial(
    pl.kernel,
    out_type=jax.ShapeDtypeStruct((NC, NP, D), jnp.float32),
    mesh=_sc_mesh,
    scratch_types=[
        pltpu.VMEM_SHARED((NP, D), jnp.float32),
        pltpu.VMEM((NSLAB, CHUNK), jnp.int32),
        pltpu.VMEM((NSLAB, CHUNK), jnp.int32),
        [pltpu.VMEM((CHUNK, D), jnp.float32) for _ in range(NBUF)],
        [pltpu.SemaphoreType.DMA for _ in range(NBUF)],
    ],
)
def _sc_scatter_add(h_hbm, src_hbm, dst_hbm, zeros_hbm, out_hbm,
                    acc_sh, sidx_v, didx_v, rows, sems):
    cid = lax.axis_index("c")
    sid = lax.axis_index("s")
    wid = sid * NC + cid

    pltpu.sync_copy(zeros_hbm.at[pl.ds(sid * ROWS_PER_TILE, ROWS_PER_TILE)],
                    acc_sh.at[pl.ds(sid * ROWS_PER_TILE, ROWS_PER_TILE)])
    plsc.subcore_barrier()

    def start(c, b):
        pltpu.async_copy(h_hbm.at[sidx_v.at[c]], rows[b], sems[b])

    def finish(c, b):
        pltpu.make_async_copy(h_hbm.at[sidx_v.at[c]], rows[b], sems[b]).wait()
        pltpu.sync_copy(rows[b], acc_sh.at[didx_v.at[c]], add=True)

    for p in range(N_CHUNKS // NSLAB):
        row0 = wid * N_CHUNKS + p * NSLAB
        pltpu.sync_copy(src_hbm.at[pl.ds(row0, NSLAB)], sidx_v)
        pltpu.sync_copy(dst_hbm.at[pl.ds(row0, NSLAB)], didx_v)

        for b in range(NBUF):
            start(b, b)

        def body(j, carry):
            c = j * NBUF
            for b in range(NBUF):
                finish(c + b, b)
                start(c + NBUF + b, b)
            return carry

        lax.fori_loop(0, NSLAB // NBUF - 1, body, 0)
        c0 = NSLAB - NBUF
        for b in range(NBUF):
            finish(c0 + b, b)

    plsc.subcore_barrier()

    pltpu.sync_copy(acc_sh.at[pl.ds(sid * ROWS_PER_TILE, ROWS_PER_TILE)],
                    out_hbm.at[cid].at[pl.ds(sid * ROWS_PER_TILE, ROWS_PER_TILE)])


@functools.partial(
    pl.kernel,
    out_type=jax.ShapeDtypeStruct((G_TOTAL, D), jnp.float32),
    mesh=_sc_mesh,
    scratch_types=[
        pltpu.VMEM_SHARED((NP, D), jnp.float32),
        pltpu.VMEM((G_PER_W,), jnp.int32),
        [pltpu.VMEM((CHUNK, D), jnp.float32) for _ in range(2)],
        [pltpu.SemaphoreType.DMA for _ in range(2)],
    ],
)
def _sc_gather(h_hbm, idx_hbm, out_hbm, h_sh, idx_v, rows, sems):
    cid = lax.axis_index("c")
    sid = lax.axis_index("s")
    wid = sid * NC + cid
    base = wid * G_PER_W

    slab = pl.ds(sid * ROWS_PER_TILE, ROWS_PER_TILE)
    pltpu.sync_copy(h_hbm.at[slab], h_sh.at[slab])
    pltpu.sync_copy(idx_hbm.at[pl.ds(base, G_PER_W)], idx_v)
    plsc.subcore_barrier()

    def start(c, b):
        pltpu.async_copy(h_sh.at[idx_v.at[pl.ds(c * CHUNK, CHUNK)]],
                         rows[b], sems[b])

    def finish(c, b):
        pltpu.make_async_copy(h_sh.at[idx_v.at[pl.ds(c * CHUNK, CHUNK)]],
                              rows[b], sems[b]).wait()
        pltpu.sync_copy(rows[b], out_hbm.at[pl.ds(base + c * CHUNK, CHUNK)])

    NB = 2
    for b in range(NB):
        start(b, b)

    def body(j, carry):
        c = j * NB
        for b in range(NB):
            finish(c + b, b)
            start(c + NB + b, b)
        return carry

    lax.fori_loop(0, G_CHUNKS // NB - 1, body, 0)
    c0 = G_CHUNKS - NB
    for b in range(NB):
        finish(c0 + b, b)


def _tc_mlp_body(h_ref, a0_ref, a1_ref, w1_ref, b1_ref, w2_ref, b2_ref, o_ref):
    z = h_ref[...] + a0_ref[0] + a1_ref[0]
    y = jnp.maximum(
        jnp.dot(z, w1_ref[...], preferred_element_type=jnp.float32)
        + b1_ref[...], 0.0)
    o = jnp.dot(y, w2_ref[...], preferred_element_type=jnp.float32) + b2_ref[...]
    o_ref[...] = jnp.maximum(o, 0.0)


def _tc_mlp(h, agg, w1, b1, w2, b2):
    blk = 1000
    grid = (N // blk,)
    return pl.pallas_call(
        _tc_mlp_body,
        grid=grid,
        in_specs=[
            pl.BlockSpec((blk, D), lambda i: (i, 0)),
            pl.BlockSpec((1, blk, D), lambda i: (0, i, 0)),
            pl.BlockSpec((1, blk, D), lambda i: (1, i, 0)),
            pl.BlockSpec((D, D), lambda i: (0, 0)),
            pl.BlockSpec((1, D), lambda i: (0, 0)),
            pl.BlockSpec((D, D), lambda i: (0, 0)),
            pl.BlockSpec((1, D), lambda i: (0, 0)),
        ],
        out_specs=pl.BlockSpec((blk, D), lambda i: (i, 0)),
        out_shape=jax.ShapeDtypeStruct((N, D), jnp.float32),
    )(h, agg, agg, w1, b1.reshape(1, D), w2, b2.reshape(1, D))


def _tc_pred_body(a_ref, b_ref, w1_ref, b1_ref, w2_ref, b2_ref, o_ref):
    p = a_ref[...] * b_ref[...]
    y = jnp.maximum(
        jnp.dot(p, w1_ref[...], preferred_element_type=jnp.float32)
        + b1_ref[...], 0.0)
    o_ref[...] = (jnp.dot(y, w2_ref[...], preferred_element_type=jnp.float32)
                  + b2_ref[...])


def _tc_pred(g, wp1, bp1, wp2, bp2):
    blk = 1024
    nblk = PE_PAD // blk
    grid = (2 * nblk,)
    return pl.pallas_call(
        _tc_pred_body,
        grid=grid,
        in_specs=[
            pl.BlockSpec((blk, D), lambda i: ((i // nblk) * 2 * nblk + i % nblk, 0)),
            pl.BlockSpec((blk, D), lambda i: ((i // nblk) * 2 * nblk + nblk + i % nblk, 0)),
            pl.BlockSpec((D, D // 2), lambda i: (0, 0)),
            pl.BlockSpec((1, D // 2), lambda i: (0, 0)),
            pl.BlockSpec((D // 2, 1), lambda i: (0, 0)),
            pl.BlockSpec((1, 1), lambda i: (0, 0)),
        ],
        out_specs=pl.BlockSpec((blk, 1), lambda i: (i, 0)),
        out_shape=jax.ShapeDtypeStruct((2 * PE_PAD, 1), jnp.float32),
    )(g, g, wp1, bp1.reshape(1, D // 2), wp2, bp2.reshape(1, 1))


def kernel(x, e, edge_index0, edge_index1, pos_edges, neg_edges,
           W1_0, b1_0, W2_0, b2_0, W1_1, b1_1, W2_1, b2_1,
           Wp1, bp1, Wp2, bp2):
    del e
    zeros = jnp.zeros((NP, D), jnp.float32)
    pad_src = jnp.zeros((E_PAD - E,), jnp.int32)
    pad_dst = jnp.full((E_PAD - E,), DUMMY_ROW, jnp.int32)

    src0 = jnp.concatenate([edge_index0[0], pad_src]).reshape(-1, CHUNK)
    dst0 = jnp.concatenate([edge_index0[1], pad_dst]).reshape(-1, CHUNK)
    src1 = jnp.concatenate([edge_index1[0], pad_src]).reshape(-1, CHUNK)
    dst1 = jnp.concatenate([edge_index1[1], pad_dst]).reshape(-1, CHUNK)

    agg0 = _sc_scatter_add(x, src0, dst0, zeros)
    h1 = _tc_mlp(x, agg0, W1_0, b1_0, W2_0, b2_0)

    agg1 = _sc_scatter_add(h1, src1, dst1, zeros)
    h2 = _tc_mlp(h1, agg1, W1_1, b1_1, W2_1, b2_1)

    gpad = jnp.zeros((PE_PAD - PE,), jnp.int32)
    idx_all = jnp.concatenate([
        pos_edges[0], gpad, pos_edges[1], gpad,
        neg_edges[0], gpad, neg_edges[1], gpad,
    ])
    h2_pad = jnp.concatenate([h2, jnp.zeros((NP - N, D), jnp.float32)])
    g = _sc_gather(h2_pad, idx_all)
    pred = _tc_pred(g, Wp1, bp1, Wp2, bp2)
    h_pos = pred[:PE]
    h_neg = pred[PE_PAD:PE_PAD + PE]
    return (h_pos, h_neg, h2)
